# baseline (speedup 1.0000x reference)
import jax
import jax.numpy as jnp
from jax import lax
from jax.experimental import pallas as pl
from jax.experimental.pallas import tpu as pltpu
from jax.experimental.pallas import tpu_sc as plsc

V = 1000000
D = 64
B = 4096
L = 200
H = 100
C = 2

LP = 224
CHUNK = LP // 2
CW = 128
NW = 32
SPW = B // NW
RPW = 2 * SPW
NCH = RPW


def _sc_pool_body(d2_hbm, m2_hbm, emb_hbm, out_hbm, idxp, msk, rowsA, rowsB,
                  accv, sem0, sem1):
    wid = lax.axis_index("s") * 2 + lax.axis_index("c")
    rbase = wid * RPW
    sbase = wid * SPW

    pltpu.sync_copy(d2_hbm.at[pl.ds(rbase, RPW)], idxp)
    pltpu.sync_copy(m2_hbm.at[pl.ds(rbase, RPW)], msk)

    def sel_body(c, carry):
        for j in range(CW // 16):
            sl = pl.ds(j * 16, 16)
            d = idxp[c, sl]
            live = msk[c, sl] != 0
            idxp[c, sl] = jnp.where(live, d >> 1, 0)
            msk[c, sl] = jnp.where(live, (d & 1) * 64, 0)
        return carry

    lax.fori_loop(0, RPW, sel_body, 0)

    pltpu.async_copy(emb_hbm.at[idxp.at[0, pl.ds(0, CHUNK)]], rowsA, sem0)

    def seq_body(s, carry):
        acc = [jnp.zeros((16,), jnp.float32) for _ in range(4)]
        for half in range(2):
            c = 2 * s + half
            buf, sem = (rowsA, sem0) if half == 0 else (rowsB, sem1)
            nbuf, nsem = (rowsB, sem1) if half == 0 else (rowsA, sem0)

            pltpu.make_async_copy(
                emb_hbm.at[idxp.at[c, pl.ds(0, CHUNK)]], buf, sem).wait()

            @pl.when(c + 1 < NCH)
            def _():
                pltpu.async_copy(
                    emb_hbm.at[idxp.at[c + 1, pl.ds(0, CHUNK)]], nbuf, nsem)

            def sum16(r, a):
                a0, a1, a2, a3 = a
                base = r * 16
                hv = msk[c, pl.ds(base, 16)]
                for rr in range(16):
                    row = base + rr
                    off = hv[rr]
                    a0 = a0 + buf[row, pl.ds(off, 16)]
                    a1 = a1 + buf[row, pl.ds(off + 16, 16)]
                    a2 = a2 + buf[row, pl.ds(off + 32, 16)]
                    a3 = a3 + buf[row, pl.ds(off + 48, 16)]
                return (a0, a1, a2, a3)

            acc = list(lax.fori_loop(0, CHUNK // 16, sum16, tuple(acc)))

        for dd in range(4):
            accv[s, pl.ds(dd * 16, 16)] = acc[dd]
        return carry

    lax.fori_loop(0, SPW, seq_body, 0)

    pltpu.sync_copy(accv, out_hbm.at[pl.ds(sbase, SPW)])


@jax.jit
def _sc_pool(d2, m2, embp):
    mesh = plsc.VectorSubcoreMesh(core_axis_name="c", subcore_axis_name="s")
    return pl.kernel(
        _sc_pool_body,
        mesh=mesh,
        out_type=jax.ShapeDtypeStruct((B, D), jnp.float32),
        scratch_types=[
            pltpu.VMEM((RPW, CW), jnp.int32),
            pltpu.VMEM((RPW, CW), jnp.int32),
            pltpu.VMEM((CHUNK, 2 * D), jnp.float32),
            pltpu.VMEM((CHUNK, 2 * D), jnp.float32),
            pltpu.VMEM((SPW, D), jnp.float32),
            pltpu.SemaphoreType.DMA,
            pltpu.SemaphoreType.DMA,
        ],
    )(d2, m2, embp)


def _mlp_body(acc_ref, mask_ref, e0_ref, w0_ref, b0_ref, w1_ref, b1_ref,
              wc_ref, bc_ref, out_ref):
    msum = jnp.sum(mask_ref[...].astype(jnp.float32), axis=1, keepdims=True)
    pooled = (acc_ref[...] - (LP - msum) * e0_ref[...]) * (1.0 / L)
    h = jnp.dot(pooled, w0_ref[...], preferred_element_type=jnp.float32)
    h = jnp.maximum(h + b0_ref[...], 0.0)
    h = jnp.dot(h, w1_ref[...], preferred_element_type=jnp.float32)
    h = jnp.maximum(h + b1_ref[...], 0.0)
    out = jnp.dot(h, wc_ref[...], preferred_element_type=jnp.float32)
    out_ref[...] = out + bc_ref[...]


@jax.jit
def _tc_mlp(acc, mask, e0, w0t, b0, w1t, b1, wct, bc):
    bt = 1024
    grid = (B // bt,)
    full = lambda shape: pl.BlockSpec(shape, lambda i: (0, 0))
    return pl.pallas_call(
        _mlp_body,
        grid=grid,
        in_specs=[
            pl.BlockSpec((bt, D), lambda i: (i, 0)),
            pl.BlockSpec((bt, L), lambda i: (i, 0)),
            full((1, D)),
            full((D, H)),
            full((1, H)),
            full((H, H)),
            full((1, H)),
            full((H, C)),
            full((1, C)),
        ],
        out_specs=pl.BlockSpec((bt, C), lambda i: (i, 0)),
        out_shape=jax.ShapeDtypeStruct((B, C), jnp.float32),
    )(acc, mask, e0, w0t, b0, w1t, b1, wct, bc)


def kernel(data, mask, emb, W0, b0, W1, b1, Wc, bc):
    dp = jnp.pad(data.astype(jnp.int32), ((0, 0), (0, LP - L)))
    mp = jnp.pad(mask, ((0, 0), (0, LP - L)))
    d2 = jnp.pad(dp.reshape(B * 2, CHUNK), ((0, 0), (0, CW - CHUNK)))
    m2 = jnp.pad(mp.reshape(B * 2, CHUNK), ((0, 0), (0, CW - CHUNK)))
    embp = emb.reshape(V // 2, 2 * D)
    acc = _sc_pool(d2, m2, embp)
    out = _tc_mlp(acc, mask, emb[0:1, :], W0.T, b0[None, :], W1.T,
                  b1[None, :], Wc.T, bc[None, :])
    return out


# speedup vs baseline: 1.9363x; 1.9363x over previous
import jax
import jax.numpy as jnp
from jax import lax
from jax.experimental import pallas as pl
from jax.experimental.pallas import tpu as pltpu
from jax.experimental.pallas import tpu_sc as plsc

V = 1000000
D = 64
B = 4096
L = 200
H = 100
C = 2

LP = 224
CHUNK = LP // 2
NW = 32
SPW = B // NW
RPW = 2 * SPW
NCH = RPW
KBUF = 8


def _sc_pool_body(d2_hbm, m2_hbm, emb_hbm, out_hbm, idxp, msk, accv, *rest):
    bufs = rest[:KBUF]
    sems = rest[KBUF:]
    wid = lax.axis_index("s") * 2 + lax.axis_index("c")
    rbase = wid * RPW
    sbase = wid * SPW

    pltpu.sync_copy(d2_hbm.at[pl.ds(rbase, RPW)], idxp)
    pltpu.sync_copy(m2_hbm.at[pl.ds(rbase, RPW)], msk)

    def sel_body(c, carry):
        for j in range(CHUNK // 16):
            sl = pl.ds(j * 16, 16)
            idxp[c, sl] = jnp.where(msk[c, sl] != 0, idxp[c, sl], 0)
        return carry

    lax.fori_loop(0, RPW, sel_body, 0)

    for k in range(KBUF):
        pltpu.async_copy(emb_hbm.at[idxp.at[k]], bufs[k], sems[k])

    def group_body(g, carry):
        base_c = g * KBUF
        for k2 in range(KBUF // 2):
            acc = [jnp.zeros((16,), jnp.float32) for _ in range(4)]
            for half in range(2):
                k = 2 * k2 + half
                c = base_c + k
                buf, sem = bufs[k], sems[k]

                pltpu.make_async_copy(emb_hbm.at[idxp.at[c]], buf, sem).wait()

                @pl.when(c + KBUF < NCH)
                def _():
                    pltpu.async_copy(
                        emb_hbm.at[idxp.at[c + KBUF]], buf, sem)

                def sum16(r, a):
                    a0, a1, a2, a3 = a
                    rb = r * 16
                    for rr in range(16):
                        row = rb + rr
                        a0 = a0 + buf[row, pl.ds(0, 16)]
                        a1 = a1 + buf[row, pl.ds(16, 16)]
                        a2 = a2 + buf[row, pl.ds(32, 16)]
                        a3 = a3 + buf[row, pl.ds(48, 16)]
                    return (a0, a1, a2, a3)

                acc = list(lax.fori_loop(0, CHUNK // 16, sum16, tuple(acc)))

            s = g * (KBUF // 2) + k2
            for dd in range(4):
                accv[s, pl.ds(dd * 16, 16)] = acc[dd]
        return carry

    lax.fori_loop(0, NCH // KBUF, group_body, 0)

    pltpu.sync_copy(accv, out_hbm.at[pl.ds(sbase, SPW)])


@jax.jit
def _sc_pool(d2, m2, emb):
    mesh = plsc.VectorSubcoreMesh(core_axis_name="c", subcore_axis_name="s")
    return pl.kernel(
        _sc_pool_body,
        mesh=mesh,
        compiler_params=pltpu.CompilerParams(use_tc_tiling_on_sc=False),
        out_type=jax.ShapeDtypeStruct((B, D), jnp.float32),
        scratch_types=(
            [
                pltpu.VMEM((RPW, CHUNK), jnp.int32),
                pltpu.VMEM((RPW, CHUNK), jnp.int32),
                pltpu.VMEM((SPW, D), jnp.float32),
            ]
            + [pltpu.VMEM((CHUNK, D), jnp.float32) for _ in range(KBUF)]
            + [pltpu.SemaphoreType.DMA for _ in range(KBUF)]
        ),
    )(d2, m2, emb)


def _mlp_body(acc_ref, mask_ref, e0_ref, w0_ref, b0_ref, w1_ref, b1_ref,
              wc_ref, bc_ref, out_ref):
    msum = jnp.sum(mask_ref[...].astype(jnp.float32), axis=1, keepdims=True)
    pooled = (acc_ref[...] - (LP - msum) * e0_ref[...]) * (1.0 / L)
    h = jnp.dot(pooled, w0_ref[...], preferred_element_type=jnp.float32)
    h = jnp.maximum(h + b0_ref[...], 0.0)
    h = jnp.dot(h, w1_ref[...], preferred_element_type=jnp.float32)
    h = jnp.maximum(h + b1_ref[...], 0.0)
    out = jnp.dot(h, wc_ref[...], preferred_element_type=jnp.float32)
    out_ref[...] = out + bc_ref[...]


@jax.jit
def _tc_mlp(acc, mask, e0, w0t, b0, w1t, b1, wct, bc):
    bt = 1024
    grid = (B // bt,)
    full = lambda shape: pl.BlockSpec(shape, lambda i: (0, 0))
    return pl.pallas_call(
        _mlp_body,
        grid=grid,
        in_specs=[
            pl.BlockSpec((bt, D), lambda i: (i, 0)),
            pl.BlockSpec((bt, L), lambda i: (i, 0)),
            full((1, D)),
            full((D, H)),
            full((1, H)),
            full((H, H)),
            full((1, H)),
            full((H, C)),
            full((1, C)),
        ],
        out_specs=pl.BlockSpec((bt, C), lambda i: (i, 0)),
        out_shape=jax.ShapeDtypeStruct((B, C), jnp.float32),
    )(acc, mask, e0, w0t, b0, w1t, b1, wct, bc)


def kernel(data, mask, emb, W0, b0, W1, b1, Wc, bc):
    dp = jnp.pad(data.astype(jnp.int32), ((0, 0), (0, LP - L)))
    mp = jnp.pad(mask, ((0, 0), (0, LP - L)))
    d2 = dp.reshape(B * 2, CHUNK)
    m2 = mp.reshape(B * 2, CHUNK)
    acc = _sc_pool(d2, m2, emb)
    out = _tc_mlp(acc, mask, emb[0:1, :], W0.T, b0[None, :], W1.T,
                  b1[None, :], Wc.T, bc[None, :])
    return out
